# baseline (device time: 12378 ns/iter reference)
import jax
import jax.numpy as jnp
from jax import lax
from jax.experimental import pallas as pl
from jax.experimental.pallas import tpu as pltpu

_RM = 256


def kernel(x, dy, gamma):
    m, d = x.shape

    def body(
        x_hbm, dy_hbm, out_ref,
        xb, dyb, grp, ybuf, local_sems,
        sendg, recvg, sendy, recvy,
    ):
        my_x = lax.axis_index("x")
        my_y = lax.axis_index("y")
        my_z = lax.axis_index("z")
        g = my_x * 4 + my_z
        r0 = g * _RM

        g_peer = [
            (((g + p) % 8) // 4, my_y, ((g + p) % 8) % 4) for p in range(1, 8)
        ]
        y_peer = (my_x, 1 - my_y, my_z)

        barrier = pltpu.get_barrier_semaphore()
        for tgt in g_peer + [y_peer]:
            pl.semaphore_signal(
                barrier, inc=1, device_id=tgt,
                device_id_type=pl.DeviceIdType.MESH,
            )

        cp_x = pltpu.make_async_copy(
            x_hbm.at[pl.ds(r0, _RM), :], xb, local_sems.at[0]
        )
        cp_dy = pltpu.make_async_copy(
            dy_hbm.at[pl.ds(r0, _RM), :], dyb, local_sems.at[1]
        )
        cp_x.start()
        cp_dy.start()

        cp_x.wait()
        xv = xb[:, :]
        mu = jnp.mean(xv, axis=1, keepdims=True)
        xc = xv - mu
        var = jnp.mean(xc * xc, axis=1, keepdims=True)
        xhat = xc * lax.rsqrt(var + 1e-5)

        cp_dy.wait()
        dyv = dyb[:, :]
        grp[g, 0, :] = jnp.sum(dyv * xhat, axis=0)
        grp[g, 1, :] = jnp.sum(dyv, axis=0)

        pl.semaphore_wait(barrier, 8)

        sends = []
        for p in range(1, 8):
            rdma = pltpu.make_async_remote_copy(
                src_ref=grp.at[g],
                dst_ref=grp.at[g],
                send_sem=sendg.at[p],
                recv_sem=recvg.at[8 - p],
                device_id=g_peer[p - 1],
                device_id_type=pl.DeviceIdType.MESH,
            )
            rdma.start()
            sends.append(rdma)

        for q in range(1, 8):
            recv = pltpu.make_async_remote_copy(
                src_ref=grp.at[g],
                dst_ref=grp.at[(g + q) % 8],
                send_sem=sendg.at[q],
                recv_sem=recvg.at[q],
                device_id=(my_x, my_y, my_z),
                device_id_type=pl.DeviceIdType.MESH,
            )
            recv.wait_recv()

        ybuf[0, :, :] = jnp.sum(grp[:, :, :], axis=0)

        yx = pltpu.make_async_remote_copy(
            src_ref=ybuf.at[0],
            dst_ref=ybuf.at[1],
            send_sem=sendy,
            recv_sem=recvy,
            device_id=y_peer,
            device_id_type=pl.DeviceIdType.MESH,
        )
        yx.start()
        yx.wait()

        out_ref[:, :] = ybuf[0, :, :] + ybuf[1, :, :]

        for rdma in sends:
            rdma.wait_send()

    return pl.pallas_call(
        body,
        out_shape=jax.ShapeDtypeStruct((2, d), jnp.float32),
        in_specs=[
            pl.BlockSpec(memory_space=pltpu.MemorySpace.HBM),
            pl.BlockSpec(memory_space=pltpu.MemorySpace.HBM),
        ],
        out_specs=pl.BlockSpec(memory_space=pltpu.VMEM),
        scratch_shapes=[
            pltpu.VMEM((_RM, d), jnp.float32),
            pltpu.VMEM((_RM, d), jnp.float32),
            pltpu.VMEM((8, 2, d), jnp.float32),
            pltpu.VMEM((2, 2, d), jnp.float32),
            pltpu.SemaphoreType.DMA((2,)),
            pltpu.SemaphoreType.DMA((8,)),
            pltpu.SemaphoreType.DMA((8,)),
            pltpu.SemaphoreType.DMA,
            pltpu.SemaphoreType.DMA,
        ],
        compiler_params=pltpu.CompilerParams(collective_id=0),
    )(
        pltpu.with_memory_space_constraint(x, pltpu.MemorySpace.HBM),
        pltpu.with_memory_space_constraint(dy, pltpu.MemorySpace.HBM),
    )


# device time: 11465 ns/iter; 1.0796x vs baseline; 1.0796x over previous
import jax
import jax.numpy as jnp
from jax import lax
from jax.experimental import pallas as pl
from jax.experimental.pallas import tpu as pltpu

N_DEV = 16
_RM = 256


def kernel(x, dy, gamma):
    m, d = x.shape

    def body(
        x_hbm, dy_hbm, out_ref,
        xb, dyb, allp, local_sems, send_sems, recv_sems,
    ):
        my_x = lax.axis_index("x")
        my_y = lax.axis_index("y")
        my_z = lax.axis_index("z")
        my_id = my_x * 8 + my_y * 4 + my_z
        rank = my_x * 4 + my_z
        r0 = rank * _RM

        barrier = pltpu.get_barrier_semaphore()
        for p in range(1, N_DEV):
            t = (my_id + p) % N_DEV
            pl.semaphore_signal(
                barrier, inc=1,
                device_id=(t // 8, (t % 8) // 4, t % 4),
                device_id_type=pl.DeviceIdType.MESH,
            )

        cp_x = pltpu.make_async_copy(
            x_hbm.at[pl.ds(r0, _RM), :], xb, local_sems.at[0]
        )
        cp_dy = pltpu.make_async_copy(
            dy_hbm.at[pl.ds(r0, _RM), :], dyb, local_sems.at[1]
        )
        cp_x.start()
        cp_dy.start()

        cp_x.wait()
        xv = xb[:, :]
        mu = jnp.mean(xv, axis=1, keepdims=True)
        xc = xv - mu
        var = jnp.mean(xc * xc, axis=1, keepdims=True)
        xhat = xc * lax.rsqrt(var + 1e-5)

        cp_dy.wait()
        dyv = dyb[:, :]
        allp[my_id, 0, :] = jnp.sum(dyv * xhat, axis=0)
        allp[my_id, 1, :] = jnp.sum(dyv, axis=0)

        pl.semaphore_wait(barrier, N_DEV - 1)

        sends = []
        for p in range(1, N_DEV):
            t = (my_id + p) % N_DEV
            rdma = pltpu.make_async_remote_copy(
                src_ref=allp.at[my_id],
                dst_ref=allp.at[my_id],
                send_sem=send_sems.at[p],
                recv_sem=recv_sems.at[N_DEV - p],
                device_id=(t // 8, (t % 8) // 4, t % 4),
                device_id_type=pl.DeviceIdType.MESH,
            )
            rdma.start()
            sends.append(rdma)

        for q in range(1, N_DEV):
            j = (my_id + q) % N_DEV
            recv = pltpu.make_async_remote_copy(
                src_ref=allp.at[my_id],
                dst_ref=allp.at[j],
                send_sem=send_sems.at[q],
                recv_sem=recv_sems.at[q],
                device_id=(my_x, my_y, my_z),
                device_id_type=pl.DeviceIdType.MESH,
            )
            recv.wait_recv()

        out_ref[:, :] = jnp.sum(allp[:, :, :], axis=0)

        for rdma in sends:
            rdma.wait_send()

    return pl.pallas_call(
        body,
        out_shape=jax.ShapeDtypeStruct((2, d), jnp.float32),
        in_specs=[
            pl.BlockSpec(memory_space=pltpu.MemorySpace.HBM),
            pl.BlockSpec(memory_space=pltpu.MemorySpace.HBM),
        ],
        out_specs=pl.BlockSpec(memory_space=pltpu.VMEM),
        scratch_shapes=[
            pltpu.VMEM((_RM, d), jnp.float32),
            pltpu.VMEM((_RM, d), jnp.float32),
            pltpu.VMEM((N_DEV, 2, d), jnp.float32),
            pltpu.SemaphoreType.DMA((2,)),
            pltpu.SemaphoreType.DMA((N_DEV,)),
            pltpu.SemaphoreType.DMA((N_DEV,)),
        ],
        compiler_params=pltpu.CompilerParams(collective_id=0),
    )(
        pltpu.with_memory_space_constraint(x, pltpu.MemorySpace.HBM),
        pltpu.with_memory_space_constraint(dy, pltpu.MemorySpace.HBM),
    )


# device time: 11383 ns/iter; 1.0874x vs baseline; 1.0072x over previous
import jax
import jax.numpy as jnp
from jax import lax
from jax.experimental import pallas as pl
from jax.experimental.pallas import tpu as pltpu

N_DEV = 16
_RM = 256


def kernel(x, dy, gamma):
    m, d = x.shape

    def body(
        x_hbm, dy_hbm, out_ref,
        xb, dyb, allp, local_sems, send_sems, recv_sems,
    ):
        my_x = lax.axis_index("x")
        my_y = lax.axis_index("y")
        my_z = lax.axis_index("z")
        my_id = my_x * 8 + my_y * 4 + my_z
        rank = my_x * 4 + my_z
        r0 = rank * _RM

        barrier = pltpu.get_barrier_semaphore()
        for p in range(1, N_DEV):
            t = (my_id + p) % N_DEV
            pl.semaphore_signal(
                barrier, inc=1,
                device_id=(t // 8, (t % 8) // 4, t % 4),
                device_id_type=pl.DeviceIdType.MESH,
            )

        half = _RM // 2
        copies = []
        for b in range(2):
            rows = pl.ds(r0 + b * half, half)
            dst = pl.ds(b * half, half)
            cx = pltpu.make_async_copy(
                x_hbm.at[rows, :], xb.at[dst, :], local_sems.at[2 * b]
            )
            cdy = pltpu.make_async_copy(
                dy_hbm.at[rows, :], dyb.at[dst, :], local_sems.at[2 * b + 1]
            )
            cx.start()
            cdy.start()
            copies.append((cx, cdy))

        dg = jnp.zeros((d,), jnp.float32)
        db = jnp.zeros((d,), jnp.float32)
        for b in range(2):
            cx, cdy = copies[b]
            cx.wait()
            xv = xb[pl.ds(b * half, half), :]
            mu = jnp.mean(xv, axis=1, keepdims=True)
            xc = xv - mu
            var = jnp.mean(xc * xc, axis=1, keepdims=True)
            xhat = xc * lax.rsqrt(var + 1e-5)
            cdy.wait()
            dyv = dyb[pl.ds(b * half, half), :]
            dg = dg + jnp.sum(dyv * xhat, axis=0)
            db = db + jnp.sum(dyv, axis=0)
        allp[my_id, 0, :] = dg
        allp[my_id, 1, :] = db

        pl.semaphore_wait(barrier, N_DEV - 1)

        sends = []
        for p in range(1, N_DEV):
            t = (my_id + p) % N_DEV
            rdma = pltpu.make_async_remote_copy(
                src_ref=allp.at[my_id],
                dst_ref=allp.at[my_id],
                send_sem=send_sems.at[p],
                recv_sem=recv_sems.at[N_DEV - p],
                device_id=(t // 8, (t % 8) // 4, t % 4),
                device_id_type=pl.DeviceIdType.MESH,
            )
            rdma.start()
            sends.append(rdma)

        for q in range(1, N_DEV):
            j = (my_id + q) % N_DEV
            recv = pltpu.make_async_remote_copy(
                src_ref=allp.at[my_id],
                dst_ref=allp.at[j],
                send_sem=send_sems.at[q],
                recv_sem=recv_sems.at[q],
                device_id=(my_x, my_y, my_z),
                device_id_type=pl.DeviceIdType.MESH,
            )
            recv.wait_recv()

        out_ref[:, :] = jnp.sum(allp[:, :, :], axis=0)

        for rdma in sends:
            rdma.wait_send()

    return pl.pallas_call(
        body,
        out_shape=jax.ShapeDtypeStruct((2, d), jnp.float32),
        in_specs=[
            pl.BlockSpec(memory_space=pltpu.MemorySpace.HBM),
            pl.BlockSpec(memory_space=pltpu.MemorySpace.HBM),
        ],
        out_specs=pl.BlockSpec(memory_space=pltpu.VMEM),
        scratch_shapes=[
            pltpu.VMEM((_RM, d), jnp.float32),
            pltpu.VMEM((_RM, d), jnp.float32),
            pltpu.VMEM((N_DEV, 2, d), jnp.float32),
            pltpu.SemaphoreType.DMA((4,)),
            pltpu.SemaphoreType.DMA((N_DEV,)),
            pltpu.SemaphoreType.DMA((N_DEV,)),
        ],
        compiler_params=pltpu.CompilerParams(collective_id=0),
    )(
        pltpu.with_memory_space_constraint(x, pltpu.MemorySpace.HBM),
        pltpu.with_memory_space_constraint(dy, pltpu.MemorySpace.HBM),
    )
